# baseline (speedup 1.0000x reference)
import functools

import jax
import jax.numpy as jnp
from jax import lax
from jax.experimental import pallas as pl
from jax.experimental.pallas import tpu as pltpu
from jax.experimental.pallas import tpu_sc as plsc

_NUM_CORES = 2
_NUM_SUBCORES = 16
_NUM_WORKERS = _NUM_CORES * _NUM_SUBCORES

_CH = 512


@functools.lru_cache(maxsize=None)
def _make_detile(num_emb, dim):
    w_lanes = 256
    n_units = num_emb // w_lanes
    tail = num_emb - n_units * w_lanes
    per_w = -(-n_units // _NUM_WORKERS)
    mesh = plsc.VectorSubcoreMesh(core_axis_name="c", subcore_axis_name="s")

    @functools.partial(
        pl.kernel,
        out_type=jax.ShapeDtypeStruct((num_emb * dim,), jnp.float32),
        mesh=mesh,
        scratch_types=[
            [pltpu.VMEM((dim, w_lanes), jnp.float32) for _ in range(2)],
            [pltpu.VMEM((w_lanes * dim,), jnp.float32) for _ in range(2)],
            [pltpu.SemaphoreType.DMA for _ in range(2)],
            [pltpu.SemaphoreType.DMA for _ in range(2)],
        ],
        compiler_params=pltpu.CompilerParams(needs_layout_passes=False),
    )
    def detile(wt_hbm, tail_hbm, out_hbm, blocks, trows, in_sems, out_sems):
        wid = lax.axis_index("s") * _NUM_CORES + lax.axis_index("c")
        u0 = wid * per_w
        n_valid = jnp.clip(n_units - u0, 0, per_w)
        lane = lax.iota(jnp.int32, 16)
        pos_pat = lane * dim

        def start_in(i, b):
            pltpu.async_copy(
                wt_hbm.at[:, pl.ds((u0 + i) * w_lanes, w_lanes)],
                blocks[b], in_sems[b],
            )

        def start_out(i, b):
            pltpu.async_copy(
                trows[b],
                out_hbm.at[pl.ds((u0 + i) * (w_lanes * dim), w_lanes * dim)],
                out_sems[b],
            )

        def wait_out(b):
            pltpu.make_async_copy(
                trows[b], out_hbm.at[pl.ds(0, w_lanes * dim)], out_sems[b]
            ).wait()

        @pl.when(n_valid > 0)
        def _():
            start_in(0, 0)

        def unit_body(i, carry):
            @pl.when(i < n_valid)
            def _():
                for b in range(2):
                    @pl.when((i & 1) == b)
                    def _():
                        pltpu.make_async_copy(
                            wt_hbm.at[:, pl.ds(0, w_lanes)], blocks[b],
                            in_sems[b],
                        ).wait()

                        @pl.when(i + 1 < n_valid)
                        def _():
                            start_in(i + 1, 1 - b)

                        @pl.when(i >= 2)
                        def _():
                            wait_out(b)

                        @plsc.parallel_loop(0, w_lanes // 16, unroll=4)
                        def _(t16):
                            base = pos_pat + t16 * (16 * dim)
                            for c in range(dim):
                                plsc.store_scatter(
                                    trows[b], [base + c],
                                    blocks[b][c, pl.ds(t16 * 16, 16)],
                                )

                        start_out(i, b)
            return carry

        lax.fori_loop(0, per_w, unit_body, 0)
        for k in range(2):
            @pl.when(n_valid > k)
            def _():
                for b in range(2):
                    @pl.when(((n_valid - 1 - k) & 1) == b)
                    def _():
                        wait_out(b)

        if tail:
            @pl.when(wid == _NUM_WORKERS - 1)
            def _():
                pltpu.sync_copy(
                    tail_hbm,
                    out_hbm.at[pl.ds(n_units * w_lanes * dim, tail * dim)],
                )

    return detile


@functools.lru_cache(maxsize=None)
def _make_lookup(num_emb, dim, n_i, n_j):
    batch = n_i * n_j
    b_per_w = batch // _NUM_WORKERS
    n_chunks = b_per_w // _CH
    units_per_chunk = _CH // 128
    n_db = dim // 8
    n_ib = n_i // 128
    out_rows = n_j * n_db * n_ib
    mesh = plsc.VectorSubcoreMesh(core_axis_name="c", subcore_axis_name="s")

    @functools.partial(
        pl.kernel,
        out_type=jax.ShapeDtypeStruct((out_rows, 1024), jnp.float32),
        mesh=mesh,
        scratch_types=[
            pltpu.VMEM((b_per_w,), jnp.int32),
            [pltpu.VMEM((_CH, dim), jnp.float32) for _ in range(2)],
            [pltpu.VMEM((n_db * units_per_chunk, 1024), jnp.float32)
             for _ in range(2)],
            [pltpu.SemaphoreType.DMA for _ in range(2)],
            [pltpu.SemaphoreType.DMA for _ in range(2)],
        ],
        compiler_params=pltpu.CompilerParams(
            use_tc_tiling_on_sc=False, needs_layout_passes=False
        ),
    )
    def lookup(ids_hbm, table_hbm, out_hbm, idx_v, rows, tbufs, g_sems, o_sems):
        wid = lax.axis_index("s") * _NUM_CORES + lax.axis_index("c")
        u_base = wid * (b_per_w // 128)
        pltpu.sync_copy(ids_hbm.at[pl.ds(wid * b_per_w, b_per_w)], idx_v)
        n_half = dim // 16
        lane = lax.iota(jnp.int32, 16)
        col_pat = (lane & 7) * 128
        row_base = (lane >> 3) * units_per_chunk

        def start_gather(c, b):
            pltpu.async_copy(
                table_hbm.at[idx_v.at[pl.ds(c * _CH, _CH)]], rows[b],
                g_sems[b],
            )

        def wait_outs(b):
            for _ in range(n_db):
                pltpu.make_async_copy(
                    tbufs[b].at[pl.ds(0, units_per_chunk)],
                    out_hbm.at[pl.ds(0, units_per_chunk), :],
                    o_sems[b],
                ).wait()

        start_gather(0, 0)

        def chunk_body(c, carry):
            for b in range(2):
                @pl.when((c & 1) == b)
                def _():
                    pltpu.make_async_copy(
                        table_hbm.at[idx_v.at[pl.ds(0, _CH)]], rows[b],
                        g_sems[b],
                    ).wait()

                    @pl.when(c + 1 < n_chunks)
                    def _():
                        start_gather(c + 1, 1 - b)

                    @pl.when(c >= 2)
                    def _():
                        wait_outs(b)

                    for ib_l in range(units_per_chunk):
                        row_pats = [
                            row_base + (2 * h * units_per_chunk + ib_l)
                            for h in range(n_half)
                        ]

                        @plsc.parallel_loop(0, 128, unroll=4)
                        def _(ii):
                            t = ib_l * 128 + ii
                            for h in range(n_half):
                                vals = rows[b][t, pl.ds(h * 16, 16)]
                                plsc.store_scatter(
                                    tbufs[b], [row_pats[h], col_pat + ii],
                                    vals,
                                )

                    u0 = u_base + c * units_per_chunk
                    j = u0 // n_ib
                    ib0 = u0 % n_ib
                    for db in range(n_db):
                        r0 = j * (n_db * n_ib) + db * n_ib + ib0
                        pltpu.async_copy(
                            tbufs[b].at[
                                pl.ds(db * units_per_chunk, units_per_chunk)],
                            out_hbm.at[pl.ds(r0, units_per_chunk), :],
                            o_sems[b],
                        )
            return carry

        lax.fori_loop(0, n_chunks, chunk_body, 0)
        for k in range(2):
            if n_chunks > k:
                wait_outs((n_chunks - 1 - k) & 1)

    return lookup


def kernel(token_ids, weight):
    n_i, n_j = token_ids.shape
    num_emb, dim = weight.shape
    ids_t = token_ids.T.reshape(n_i * n_j).astype(jnp.int32)
    n_full = (num_emb // 128) * 128
    tail_rows = weight[n_full:, :].reshape(-1)
    table = _make_detile(num_emb, dim)(weight.T, tail_rows)
    table = table.reshape(num_emb, dim)
    out2d = _make_lookup(num_emb, dim, n_i, n_j)(ids_t, table)
    out5d = out2d.reshape(n_j, dim // 8, n_i // 128, 8, 128)
    return out5d.transpose(2, 4, 0, 1, 3).reshape(n_i, n_j, dim)


# speedup vs baseline: 1.4564x; 1.4564x over previous
import functools

import jax
import jax.numpy as jnp
from jax import lax
from jax.experimental import pallas as pl
from jax.experimental.pallas import tpu as pltpu
from jax.experimental.pallas import tpu_sc as plsc

_NUM_CORES = 2
_NUM_SUBCORES = 16
_NUM_WORKERS = _NUM_CORES * _NUM_SUBCORES

_CH = 512


@functools.lru_cache(maxsize=None)
def _make_detile(num_emb, dim):
    w_lanes = 256
    n_units = num_emb // w_lanes
    tail = num_emb - n_units * w_lanes
    per_w = -(-n_units // _NUM_WORKERS)
    mesh = plsc.VectorSubcoreMesh(core_axis_name="c", subcore_axis_name="s")

    @functools.partial(
        pl.kernel,
        out_type=jax.ShapeDtypeStruct((num_emb * dim,), jnp.float32),
        mesh=mesh,
        scratch_types=[
            [pltpu.VMEM((dim, w_lanes + 1), jnp.float32) for _ in range(2)],
            [pltpu.VMEM((w_lanes * dim,), jnp.float32) for _ in range(2)],
            [pltpu.SemaphoreType.DMA for _ in range(2)],
            [pltpu.SemaphoreType.DMA for _ in range(2)],
        ],
        compiler_params=pltpu.CompilerParams(needs_layout_passes=False),
    )
    def detile(wt_hbm, tail_hbm, out_hbm, blocks, trows, in_sems, out_sems):
        wid = lax.axis_index("s") * _NUM_CORES + lax.axis_index("c")
        u0 = wid * per_w
        n_valid = jnp.clip(n_units - u0, 0, per_w)
        lane = lax.iota(jnp.int32, 16)
        c_pats = [lane + h * 16 for h in range(dim // 16)]

        def start_in(i, b):
            pltpu.async_copy(
                wt_hbm.at[:, pl.ds((u0 + i) * w_lanes, w_lanes)],
                blocks[b].at[:, pl.ds(0, w_lanes)], in_sems[b],
            )

        def start_out(i, b):
            pltpu.async_copy(
                trows[b],
                out_hbm.at[pl.ds((u0 + i) * (w_lanes * dim), w_lanes * dim)],
                out_sems[b],
            )

        def wait_out(b):
            pltpu.make_async_copy(
                trows[b], out_hbm.at[pl.ds(0, w_lanes * dim)], out_sems[b]
            ).wait()

        @pl.when(n_valid > 0)
        def _():
            start_in(0, 0)

        def unit_body(i, carry):
            @pl.when(i < n_valid)
            def _():
                for b in range(2):
                    @pl.when((i & 1) == b)
                    def _():
                        pltpu.make_async_copy(
                            wt_hbm.at[:, pl.ds(0, w_lanes)],
                            blocks[b].at[:, pl.ds(0, w_lanes)], in_sems[b],
                        ).wait()

                        @pl.when(i + 1 < n_valid)
                        def _():
                            start_in(i + 1, 1 - b)

                        @pl.when(i >= 2)
                        def _():
                            wait_out(b)

                        @plsc.parallel_loop(0, w_lanes, unroll=4)
                        def _(t):
                            t_vec = jnp.full((16,), t, jnp.int32)
                            for h in range(dim // 16):
                                vals = plsc.load_gather(
                                    blocks[b], [c_pats[h], t_vec])
                                trows[b][pl.ds(t * dim + h * 16, 16)] = vals

                        start_out(i, b)
            return carry

        lax.fori_loop(0, per_w, unit_body, 0)
        for k in range(2):
            @pl.when(n_valid > k)
            def _():
                for b in range(2):
                    @pl.when(((n_valid - 1 - k) & 1) == b)
                    def _():
                        wait_out(b)

        if tail:
            @pl.when(wid == _NUM_WORKERS - 1)
            def _():
                pltpu.sync_copy(
                    tail_hbm,
                    out_hbm.at[pl.ds(n_units * w_lanes * dim, tail * dim)],
                )

    return detile


@functools.lru_cache(maxsize=None)
def _make_lookup(num_emb, dim, n_i, n_j):
    batch = n_i * n_j
    b_per_w = batch // _NUM_WORKERS
    n_chunks = b_per_w // _CH
    units_per_chunk = _CH // 128
    n_db = dim // 8
    n_ib = n_i // 128
    out_rows = n_j * n_db * n_ib
    mesh = plsc.VectorSubcoreMesh(core_axis_name="c", subcore_axis_name="s")

    @functools.partial(
        pl.kernel,
        out_type=jax.ShapeDtypeStruct((out_rows, 8, 128), jnp.float32),
        mesh=mesh,
        scratch_types=[
            pltpu.VMEM((b_per_w,), jnp.int32),
            [pltpu.VMEM((_CH, dim), jnp.float32) for _ in range(2)],
            [pltpu.VMEM((n_db * units_per_chunk, 8, 129), jnp.float32)
             for _ in range(2)],
            [pltpu.SemaphoreType.DMA for _ in range(2)],
            [pltpu.SemaphoreType.DMA for _ in range(2)],
        ],
        compiler_params=pltpu.CompilerParams(
            use_tc_tiling_on_sc=False, needs_layout_passes=False
        ),
    )
    def lookup(ids_hbm, table_hbm, out_hbm, idx_v, rows, tbufs, g_sems, o_sems):
        wid = lax.axis_index("s") * _NUM_CORES + lax.axis_index("c")
        u_base = wid * (b_per_w // 128)
        pltpu.sync_copy(ids_hbm.at[pl.ds(wid * b_per_w, b_per_w)], idx_v)
        n_half = dim // 16
        lane = lax.iota(jnp.int32, 16)
        di_pat = lane & 7
        row_base = (lane >> 3) * units_per_chunk

        def start_gather(c, b):
            pltpu.async_copy(
                table_hbm.at[idx_v.at[pl.ds(c * _CH, _CH)]], rows[b],
                g_sems[b],
            )

        def wait_outs(b):
            for _ in range(n_db):
                pltpu.make_async_copy(
                    tbufs[b].at[pl.ds(0, units_per_chunk), :, pl.ds(0, 128)],
                    out_hbm.at[pl.ds(0, units_per_chunk), :, :],
                    o_sems[b],
                ).wait()

        start_gather(0, 0)

        def chunk_body(c, carry):
            for b in range(2):
                @pl.when((c & 1) == b)
                def _():
                    pltpu.make_async_copy(
                        table_hbm.at[idx_v.at[pl.ds(0, _CH)]], rows[b],
                        g_sems[b],
                    ).wait()

                    @pl.when(c + 1 < n_chunks)
                    def _():
                        start_gather(c + 1, 1 - b)

                    @pl.when(c >= 2)
                    def _():
                        wait_outs(b)

                    for ib_l in range(units_per_chunk):
                        row_pats = [
                            row_base + (2 * h * units_per_chunk + ib_l)
                            for h in range(n_half)
                        ]

                        @plsc.parallel_loop(0, 128, unroll=4)
                        def _(ii):
                            t = ib_l * 128 + ii
                            ii_vec = jnp.full((16,), ii, jnp.int32)
                            for h in range(n_half):
                                vals = rows[b][t, pl.ds(h * 16, 16)]
                                plsc.store_scatter(
                                    tbufs[b], [row_pats[h], di_pat, ii_vec],
                                    vals,
                                )

                    u0 = u_base + c * units_per_chunk
                    j = u0 // n_ib
                    ib0 = u0 % n_ib
                    for db in range(n_db):
                        r0 = j * (n_db * n_ib) + db * n_ib + ib0
                        pltpu.async_copy(
                            tbufs[b].at[
                                pl.ds(db * units_per_chunk, units_per_chunk),
                                :, pl.ds(0, 128)],
                            out_hbm.at[pl.ds(r0, units_per_chunk), :, :],
                            o_sems[b],
                        )
            return carry

        lax.fori_loop(0, n_chunks, chunk_body, 0)
        for k in range(2):
            if n_chunks > k:
                wait_outs((n_chunks - 1 - k) & 1)

    return lookup


def kernel(token_ids, weight):
    n_i, n_j = token_ids.shape
    num_emb, dim = weight.shape
    ids_t = token_ids.T.reshape(n_i * n_j).astype(jnp.int32)
    n_full = (num_emb // 128) * 128
    tail_rows = weight[n_full:, :].reshape(-1)
    table = _make_detile(num_emb, dim)(weight.T, tail_rows)
    table = table.reshape(num_emb, dim)
    out2d = _make_lookup(num_emb, dim, n_i, n_j)(ids_t, table)
    out5d = out2d.reshape(n_j, dim // 8, n_i // 128, 8, 128)
    return out5d.transpose(2, 4, 0, 1, 3).reshape(n_i, n_j, dim)
